# baseline (speedup 1.0000x reference)
import functools

import numpy as np
import jax
import jax.numpy as jnp
from jax import lax
from jax.experimental import pallas as pl
from jax.experimental.pallas import tpu as pltpu
from jax.experimental.pallas import tpu_sc as plsc

N = 10000
E = 320000
HID = 128
OUT = 128
H = 8
DK = 16

NC = 2
NS = 16
NW = NC * NS
B = 64
G = B // 16
NCHUNKS = E // B
CHUNK_ITERS = -(-NCHUNKS // NW)
PAIRS = -(-CHUNK_ITERS // 2)
NPAD = 10112
ROWS_PER_SUB = NPAD // NS
ZP_ROWS = 1280
ZP_PER_SUB = ZP_ROWS // NS
ZPAD = ZP_ROWS * 8

ROW_BLK = 1000


def _build_perm() -> np.ndarray:
    idx = np.zeros(HID, dtype=np.int32)
    for h in range(H):
        for d in range(DK):
            p = (d // 2) * 16 + (h if d % 2 == 0 else 15 - h)
            idx[p] = h * DK + d
    return idx


_PERM = _build_perm()

_TILE16 = np.zeros((16, HID), dtype=np.float32)
for _c in range(8):
    for _l in range(16):
        _TILE16[_l, _c * 16 + _l] = 1.0


def _proj_body(x_ref, wq_ref, bq_ref, wk_ref, wv_ref, q_ref, k_ref, v_ref):
    x = x_ref[...]
    dn = (((1,), (1,)), ((), ()))
    q_ref[...] = lax.dot_general(x, wq_ref[...], dn,
                                 preferred_element_type=jnp.float32) + bq_ref[...]
    k_ref[...] = lax.dot_general(x, wk_ref[...], dn,
                                 preferred_element_type=jnp.float32)
    v_ref[...] = lax.dot_general(x, wv_ref[...], dn,
                                 preferred_element_type=jnp.float32)


def _project(x, wq_p, bq_p, wk_p, wv_p):
    f32 = jnp.float32
    full = lambda s: pl.BlockSpec(s, lambda i: (0, 0))
    row = pl.BlockSpec((ROW_BLK, HID), lambda i: (i, 0))
    return pl.pallas_call(
        _proj_body,
        grid=(N // ROW_BLK,),
        in_specs=[row, full((HID, HID)), full((1, HID)), full((HID, HID)),
                  full((HID, HID))],
        out_specs=[row, row, row],
        out_shape=[jax.ShapeDtypeStruct((N, HID), f32)] * 3,
    )(x, wq_p, bq_p, wk_p, wv_p)


def _sc_edges(q, k, v, edge_index):
    f32 = jnp.float32
    NSPLIT = 4
    HB = B // NSPLIT
    mesh = plsc.VectorSubcoreMesh(core_axis_name="c", subcore_axis_name="s")

    @functools.partial(
        pl.kernel,
        out_type=[jax.ShapeDtypeStruct((NC, NPAD, HID), f32),
                  jax.ShapeDtypeStruct((NC, ZP_ROWS, HID), f32)],
        mesh=mesh,
        scratch_types=[
            pltpu.VMEM((B,), jnp.int32),
            pltpu.VMEM((B,), jnp.int32),
            pltpu.VMEM((B,), jnp.int32),
            pltpu.VMEM((B,), jnp.int32),
            pltpu.VMEM((B,), jnp.int32),
            pltpu.VMEM((B, HID), f32),
            pltpu.VMEM((B, HID), f32),
            pltpu.VMEM((B, HID), f32),
            pltpu.VMEM((B, 16), f32),
            pltpu.VMEM_SHARED((NPAD, HID), f32),
            pltpu.VMEM_SHARED((ZP_ROWS, HID), f32),
            pltpu.SemaphoreType.DMA,
            pltpu.SemaphoreType.DMA,
            pltpu.SemaphoreType.DMA,
            pltpu.SemaphoreType.DMA,
        ],
    )
    def sc_kernel(q_hbm, k_hbm, v_hbm, src_hbm, dst_hbm, wv_hbm, z_hbm,
                  src0, dst0, src1, dst1, zrid, kb, vb, qb, scb,
                  wv_sh, zp_sh, si0, si1, sg, sz):
        cid = lax.axis_index("c")
        sid = lax.axis_index("s")
        wid = sid * NC + cid

        zero16 = jnp.zeros((16,), f32)

        @pl.loop(0, B)
        def _zero_fill(r):
            for c in range(8):
                qb[r, pl.ds(c * 16, 16)] = zero16

        @pl.loop(0, ROWS_PER_SUB // 8)
        def _zero_wv(j):
            pltpu.async_copy(qb.at[pl.ds(0, 8)],
                             wv_sh.at[pl.ds(sid * ROWS_PER_SUB + j * 8, 8)],
                             si0)

        @pl.loop(0, ZP_PER_SUB // 8)
        def _zero_zp(j):
            pltpu.async_copy(qb.at[pl.ds(0, 8)],
                             zp_sh.at[pl.ds(sid * ZP_PER_SUB + j * 8, 8)],
                             si0)

        @pl.loop(0, ROWS_PER_SUB // 8 + ZP_PER_SUB // 8)
        def _zero_drain(j):
            pltpu.make_async_copy(
                qb.at[pl.ds(0, 8)],
                wv_sh.at[pl.ds(0, 8)], si0).wait()

        plsc.subcore_barrier()

        def load_idx(c, srcb, dstb, sem):
            off = c * B
            pltpu.async_copy(src_hbm.at[pl.ds(off, B)], srcb, sem)
            pltpu.async_copy(dst_hbm.at[pl.ds(off, B)], dstb, sem)

        def wait_idx(srcb, dstb, sem):
            pltpu.make_async_copy(src_hbm.at[pl.ds(0, B)], srcb, sem).wait()
            pltpu.make_async_copy(dst_hbm.at[pl.ds(0, B)], dstb, sem).wait()

        def process(c_next, srcb, dstb, srcn, dstn, semn, first, last):
            @pl.when(jnp.logical_not(first))
            def _drain_z():
                pltpu.make_async_copy(qb, zp_sh.at[zrid], sz).wait()

            for (tab, idxb, dest) in ((k_hbm, srcb, kb), (v_hbm, srcb, vb),
                                      (q_hbm, dstb, qb)):
                for hh in range(NSPLIT):
                    pltpu.async_copy(
                        tab.at[idxb.at[pl.ds(hh * HB, HB)]],
                        dest.at[pl.ds(hh * HB, HB)], sg)
            @pl.when(c_next < NCHUNKS)
            def _pref():
                load_idx(c_next, srcn, dstn, semn)
            for _ in range(3 * NSPLIT):
                pltpu.make_async_copy(
                    k_hbm.at[src0.at[pl.ds(0, HB)]],
                    kb.at[pl.ds(0, HB)], sg).wait()

            @pl.loop(0, B)
            def _edge(i):
                acc = kb[i, pl.ds(0, 16)] * qb[i, pl.ds(0, 16)]
                for c in range(1, 8):
                    acc += (kb[i, pl.ds(c * 16, 16)]
                            * qb[i, pl.ds(c * 16, 16)])
                ts = (acc + lax.rev(acc, (0,))) * 0.25
                ts = jnp.minimum(jnp.maximum(ts, -5.0), 5.0)
                s = jnp.exp(ts)
                scb[i, :] = s
                for c in range(8):
                    qb[i, pl.ds(c * 16, 16)] = (
                        vb[i, pl.ds(c * 16, 16)] * s)

            pltpu.sync_copy(qb, wv_sh.at[dstb], add=True)

            @pl.loop(0, G)
            def _z_group(g):
                d16 = dstb[pl.ds(g * 16, 16)]
                zrid[pl.ds(g * 16, 16)] = lax.shift_right_logical(d16, 3)
                for t in range(16):
                    i = g * 16 + t
                    s = scb[i, :]
                    for c in range(8):
                        qb[i, pl.ds(c * 16, 16)] = zero16
                    m = lax.rem(d16[t], 8)
                    qb[i, pl.ds(m * 16, 16)] = s

            pltpu.async_copy(qb, zp_sh.at[zrid], sz)

            @pl.when(last)
            def _drain_last():
                pltpu.make_async_copy(qb, zp_sh.at[zrid], sz).wait()

        @pl.when(wid < NCHUNKS)
        def _prime():
            load_idx(wid, src0, dst0, si0)

        @pl.loop(0, PAIRS)
        def _pair(jj):
            j0 = jj * 2
            c0 = wid + j0 * NW
            c1 = wid + (j0 + 1) * NW
            c2 = wid + (j0 + 2) * NW

            @pl.when(c0 < NCHUNKS)
            def _proc0():
                wait_idx(src0, dst0, si0)
                process(c1, src0, dst0, src1, dst1, si1,
                        jj == 0, c1 >= NCHUNKS)

            @pl.when(c1 < NCHUNKS)
            def _proc1():
                wait_idx(src1, dst1, si1)
                process(c2, src1, dst1, src0, dst0, si0,
                        jnp.bool_(False), c2 >= NCHUNKS)

        plsc.subcore_barrier()

        base = sid * ROWS_PER_SUB
        pltpu.sync_copy(wv_sh.at[pl.ds(base, ROWS_PER_SUB)],
                        wv_hbm.at[cid, pl.ds(base, ROWS_PER_SUB)])
        zbase = sid * ZP_PER_SUB
        pltpu.sync_copy(zp_sh.at[pl.ds(zbase, ZP_PER_SUB)],
                        z_hbm.at[cid, pl.ds(zbase, ZP_PER_SUB)])

    return sc_kernel(q, k, v, edge_index[0], edge_index[1])


def _out_body(wv0_ref, wv1_ref, z0_ref, z1_ref, t_ref, wo_ref, bo_ref, out_ref):
    wv = wv0_ref[...] + wv1_ref[...]
    z = z0_ref[...] + z1_ref[...]
    den = lax.dot_general(z, t_ref[...], (((1,), (0,)), ((), ())),
                          preferred_element_type=jnp.float32)
    o = wv / den
    out_ref[...] = lax.dot_general(o, wo_ref[...], (((1,), (1,)), ((), ())),
                                   preferred_element_type=jnp.float32) + bo_ref[...]


def _finish(wv0, wv1, z0, z1, tile16, wo_p, bo2):
    f32 = jnp.float32
    row = pl.BlockSpec((ROW_BLK, HID), lambda i: (i, 0))
    zrow = pl.BlockSpec((ROW_BLK, 16), lambda i: (i, 0))
    full = lambda s: pl.BlockSpec(s, lambda i: (0, 0))
    return pl.pallas_call(
        _out_body,
        grid=(N // ROW_BLK,),
        in_specs=[row, row, zrow, zrow, full((16, HID)), full((OUT, HID)),
                  full((1, OUT))],
        out_specs=pl.BlockSpec((ROW_BLK, OUT), lambda i: (i, 0)),
        out_shape=jax.ShapeDtypeStruct((N, OUT), f32),
    )(wv0, wv1, z0, z1, tile16, wo_p, bo2)


def kernel(inputs, edge_index, Wq, bq, Wk, Wv, Wo, bo):
    perm = jnp.asarray(_PERM)
    wq_p = Wq[perm]
    bq_p = bq[perm].reshape(1, HID)
    wk_p = Wk[perm]
    wv_p = Wv[perm]
    wo_p = Wo[:, perm]
    tile16 = jnp.asarray(_TILE16)

    q, k, v = _project(inputs, wq_p, bq_p, wk_p, wv_p)
    wv_parts, z_parts = _sc_edges(q, k, v, edge_index)
    z_flat = z_parts.reshape(NC, ZPAD, 16)
    out = _finish(wv_parts[0], wv_parts[1], z_flat[0], z_flat[1],
                  tile16, wo_p, bo.reshape(1, OUT))
    return out


# speedup vs baseline: 1.0752x; 1.0752x over previous
import functools

import numpy as np
import jax
import jax.numpy as jnp
from jax import lax
from jax.experimental import pallas as pl
from jax.experimental.pallas import tpu as pltpu
from jax.experimental.pallas import tpu_sc as plsc

N = 10000
E = 320000
HID = 128
OUT = 128
H = 8
DK = 16

NC = 2
NS = 16
NW = NC * NS
B = 64
G = B // 16
NCHUNKS = E // B
CHUNK_ITERS = -(-NCHUNKS // NW)
PAIRS = -(-CHUNK_ITERS // 2)
NPAD = 10112
ROWS_PER_SUB = NPAD // NS
ZP_ROWS = 1280
ZP_PER_SUB = ZP_ROWS // NS
ZPAD = ZP_ROWS * 8

ROW_BLK = 1000


def _build_perm() -> np.ndarray:
    idx = np.zeros(HID, dtype=np.int32)
    for h in range(H):
        for d in range(DK):
            p = (d // 2) * 16 + (h if d % 2 == 0 else 15 - h)
            idx[p] = h * DK + d
    return idx


_PERM = _build_perm()

_TILE16 = np.zeros((16, HID), dtype=np.float32)
for _c in range(8):
    for _l in range(16):
        _TILE16[_l, _c * 16 + _l] = 1.0


def _proj_body(x_ref, wq_ref, bq_ref, wk_ref, wv_ref, q_ref, k_ref, v_ref):
    x = x_ref[...]
    dn = (((1,), (1,)), ((), ()))
    q_ref[...] = lax.dot_general(x, wq_ref[...], dn,
                                 preferred_element_type=jnp.float32) + bq_ref[...]
    k_ref[...] = lax.dot_general(x, wk_ref[...], dn,
                                 preferred_element_type=jnp.float32)
    v_ref[...] = lax.dot_general(x, wv_ref[...], dn,
                                 preferred_element_type=jnp.float32)


def _project(x, wq_p, bq_p, wk_p, wv_p):
    f32 = jnp.float32
    full = lambda s: pl.BlockSpec(s, lambda i: (0, 0))
    row = pl.BlockSpec((ROW_BLK, HID), lambda i: (i, 0))
    return pl.pallas_call(
        _proj_body,
        grid=(N // ROW_BLK,),
        in_specs=[row, full((HID, HID)), full((1, HID)), full((HID, HID)),
                  full((HID, HID))],
        out_specs=[row, row, row],
        out_shape=[jax.ShapeDtypeStruct((N, HID), f32)] * 3,
    )(x, wq_p, bq_p, wk_p, wv_p)


def _sc_edges(q, k, v, edge_index):
    f32 = jnp.float32
    NSPLIT = 4
    HB = B // NSPLIT
    mesh = plsc.VectorSubcoreMesh(core_axis_name="c", subcore_axis_name="s")

    @functools.partial(
        pl.kernel,
        out_type=[jax.ShapeDtypeStruct((NC, NPAD, HID), f32),
                  jax.ShapeDtypeStruct((NC, ZP_ROWS, HID), f32)],
        mesh=mesh,
        scratch_types=[
            pltpu.VMEM((B,), jnp.int32),
            pltpu.VMEM((B,), jnp.int32),
            pltpu.VMEM((B,), jnp.int32),
            pltpu.VMEM((B,), jnp.int32),
            pltpu.VMEM((B,), jnp.int32),
            pltpu.VMEM((B, HID), f32),
            pltpu.VMEM((B, HID), f32),
            pltpu.VMEM((B, HID), f32),
            pltpu.VMEM((B, 16), f32),
            pltpu.VMEM_SHARED((NPAD, HID), f32),
            pltpu.VMEM_SHARED((ZP_ROWS, HID), f32),
            pltpu.SemaphoreType.DMA,
            pltpu.SemaphoreType.DMA,
            pltpu.SemaphoreType.DMA,
            pltpu.SemaphoreType.DMA,
            pltpu.SemaphoreType.DMA,
        ],
    )
    def sc_kernel(q_hbm, k_hbm, v_hbm, src_hbm, dst_hbm, wv_hbm, z_hbm,
                  src0, dst0, src1, dst1, zrid, kb, vb, qb, scb,
                  wv_sh, zp_sh, si0, si1, sg, sw, sz):
        cid = lax.axis_index("c")
        sid = lax.axis_index("s")
        wid = sid * NC + cid

        zero16 = jnp.zeros((16,), f32)

        @pl.loop(0, B)
        def _zero_fill(r):
            for c in range(8):
                qb[r, pl.ds(c * 16, 16)] = zero16

        @pl.loop(0, ROWS_PER_SUB // 8)
        def _zero_wv(j):
            pltpu.async_copy(qb.at[pl.ds(0, 8)],
                             wv_sh.at[pl.ds(sid * ROWS_PER_SUB + j * 8, 8)],
                             si0)

        @pl.loop(0, ZP_PER_SUB // 8)
        def _zero_zp(j):
            pltpu.async_copy(qb.at[pl.ds(0, 8)],
                             zp_sh.at[pl.ds(sid * ZP_PER_SUB + j * 8, 8)],
                             si0)

        @pl.loop(0, ROWS_PER_SUB // 8 + ZP_PER_SUB // 8)
        def _zero_drain(j):
            pltpu.make_async_copy(
                qb.at[pl.ds(0, 8)],
                wv_sh.at[pl.ds(0, 8)], si0).wait()

        plsc.subcore_barrier()

        def load_idx(c, srcb, dstb, sem):
            off = c * B
            pltpu.async_copy(src_hbm.at[pl.ds(off, B)], srcb, sem)
            pltpu.async_copy(dst_hbm.at[pl.ds(off, B)], dstb, sem)

        def wait_idx(srcb, dstb, sem):
            pltpu.make_async_copy(src_hbm.at[pl.ds(0, B)], srcb, sem).wait()
            pltpu.make_async_copy(dst_hbm.at[pl.ds(0, B)], dstb, sem).wait()

        def process(c_next, srcb, dstb, srcn, dstn, semn, first, last):
            @pl.when(jnp.logical_not(first))
            def _drain_prev():
                pltpu.make_async_copy(qb, wv_sh.at[zrid], sw).wait()
                pltpu.make_async_copy(vb, zp_sh.at[zrid], sz).wait()

            for (tab, idxb, dest) in ((k_hbm, srcb, kb), (v_hbm, srcb, vb),
                                      (q_hbm, dstb, qb)):
                for hh in range(NSPLIT):
                    pltpu.async_copy(
                        tab.at[idxb.at[pl.ds(hh * HB, HB)]],
                        dest.at[pl.ds(hh * HB, HB)], sg)
            @pl.when(c_next < NCHUNKS)
            def _pref():
                load_idx(c_next, srcn, dstn, semn)
            for _ in range(3 * NSPLIT):
                pltpu.make_async_copy(
                    k_hbm.at[src0.at[pl.ds(0, HB)]],
                    kb.at[pl.ds(0, HB)], sg).wait()

            @pl.loop(0, B)
            def _edge(i):
                acc = kb[i, pl.ds(0, 16)] * qb[i, pl.ds(0, 16)]
                for c in range(1, 8):
                    acc += (kb[i, pl.ds(c * 16, 16)]
                            * qb[i, pl.ds(c * 16, 16)])
                ts = (acc + lax.rev(acc, (0,))) * 0.25
                ts = jnp.minimum(jnp.maximum(ts, -5.0), 5.0)
                s = jnp.exp(ts)
                scb[i, :] = s
                for c in range(8):
                    qb[i, pl.ds(c * 16, 16)] = (
                        vb[i, pl.ds(c * 16, 16)] * s)

            pltpu.async_copy(qb, wv_sh.at[dstb], sw, add=True)

            @pl.loop(0, G)
            def _z_group(g):
                d16 = dstb[pl.ds(g * 16, 16)]
                zrid[pl.ds(g * 16, 16)] = lax.shift_right_logical(d16, 3)
                for t in range(16):
                    i = g * 16 + t
                    s = scb[i, :]
                    for c in range(8):
                        vb[i, pl.ds(c * 16, 16)] = zero16
                    m = lax.rem(d16[t], 8)
                    vb[i, pl.ds(m * 16, 16)] = s

            pltpu.async_copy(vb, zp_sh.at[zrid], sz)

            @pl.when(last)
            def _drain_last():
                pltpu.make_async_copy(qb, wv_sh.at[zrid], sw).wait()
                pltpu.make_async_copy(vb, zp_sh.at[zrid], sz).wait()

        @pl.when(wid < NCHUNKS)
        def _prime():
            load_idx(wid, src0, dst0, si0)

        @pl.loop(0, PAIRS)
        def _pair(jj):
            j0 = jj * 2
            c0 = wid + j0 * NW
            c1 = wid + (j0 + 1) * NW
            c2 = wid + (j0 + 2) * NW

            @pl.when(c0 < NCHUNKS)
            def _proc0():
                wait_idx(src0, dst0, si0)
                process(c1, src0, dst0, src1, dst1, si1,
                        jj == 0, c1 >= NCHUNKS)

            @pl.when(c1 < NCHUNKS)
            def _proc1():
                wait_idx(src1, dst1, si1)
                process(c2, src1, dst1, src0, dst0, si0,
                        jnp.bool_(False), c2 >= NCHUNKS)

        plsc.subcore_barrier()

        base = sid * ROWS_PER_SUB
        pltpu.sync_copy(wv_sh.at[pl.ds(base, ROWS_PER_SUB)],
                        wv_hbm.at[cid, pl.ds(base, ROWS_PER_SUB)])
        zbase = sid * ZP_PER_SUB
        pltpu.sync_copy(zp_sh.at[pl.ds(zbase, ZP_PER_SUB)],
                        z_hbm.at[cid, pl.ds(zbase, ZP_PER_SUB)])

    return sc_kernel(q, k, v, edge_index[0], edge_index[1])


def _out_body(wv0_ref, wv1_ref, z0_ref, z1_ref, t_ref, wo_ref, bo_ref, out_ref):
    wv = wv0_ref[...] + wv1_ref[...]
    z = z0_ref[...] + z1_ref[...]
    den = lax.dot_general(z, t_ref[...], (((1,), (0,)), ((), ())),
                          preferred_element_type=jnp.float32)
    o = wv / den
    out_ref[...] = lax.dot_general(o, wo_ref[...], (((1,), (1,)), ((), ())),
                                   preferred_element_type=jnp.float32) + bo_ref[...]


def _finish(wv0, wv1, z0, z1, tile16, wo_p, bo2):
    f32 = jnp.float32
    row = pl.BlockSpec((ROW_BLK, HID), lambda i: (i, 0))
    zrow = pl.BlockSpec((ROW_BLK, 16), lambda i: (i, 0))
    full = lambda s: pl.BlockSpec(s, lambda i: (0, 0))
    return pl.pallas_call(
        _out_body,
        grid=(N // ROW_BLK,),
        in_specs=[row, row, zrow, zrow, full((16, HID)), full((OUT, HID)),
                  full((1, OUT))],
        out_specs=pl.BlockSpec((ROW_BLK, OUT), lambda i: (i, 0)),
        out_shape=jax.ShapeDtypeStruct((N, OUT), f32),
    )(wv0, wv1, z0, z1, tile16, wo_p, bo2)


def kernel(inputs, edge_index, Wq, bq, Wk, Wv, Wo, bo):
    perm = jnp.asarray(_PERM)
    wq_p = Wq[perm]
    bq_p = bq[perm].reshape(1, HID)
    wk_p = Wk[perm]
    wv_p = Wv[perm]
    wo_p = Wo[:, perm]
    tile16 = jnp.asarray(_TILE16)

    q, k, v = _project(inputs, wq_p, bq_p, wk_p, wv_p)
    wv_parts, z_parts = _sc_edges(q, k, v, edge_index)
    z_flat = z_parts.reshape(NC, ZPAD, 16)
    out = _finish(wv_parts[0], wv_parts[1], z_flat[0], z_flat[1],
                  tile16, wo_p, bo.reshape(1, OUT))
    return out


# speedup vs baseline: 1.1078x; 1.0303x over previous
import functools

import numpy as np
import jax
import jax.numpy as jnp
from jax import lax
from jax.experimental import pallas as pl
from jax.experimental.pallas import tpu as pltpu
from jax.experimental.pallas import tpu_sc as plsc

N = 10000
E = 320000
HID = 128
OUT = 128
H = 8
DK = 16

NC = 2
NS = 16
NW = NC * NS
B = 64
G = B // 16
NCHUNKS = E // B
CHUNK_ITERS = -(-NCHUNKS // NW)
PAIRS = -(-CHUNK_ITERS // 2)
NPAD = 10112
ROWS_PER_SUB = NPAD // NS
ZP_ROWS = 1280
ZP_PER_SUB = ZP_ROWS // NS
ZPAD = ZP_ROWS * 8

ROW_BLK = 1000


def _build_perm() -> np.ndarray:
    idx = np.zeros(HID, dtype=np.int32)
    for h in range(H):
        for d in range(DK):
            p = (d // 2) * 16 + (h if d % 2 == 0 else 15 - h)
            idx[p] = h * DK + d
    return idx


_PERM = _build_perm()

_TILE16 = np.zeros((16, HID), dtype=np.float32)
for _c in range(8):
    for _l in range(16):
        _TILE16[_l, _c * 16 + _l] = 1.0


def _proj_body(x_ref, wq_ref, bq_ref, wk_ref, wv_ref, q_ref, k_ref, v_ref):
    x = x_ref[...]
    dn = (((1,), (1,)), ((), ()))
    q_ref[...] = lax.dot_general(x, wq_ref[...], dn,
                                 preferred_element_type=jnp.float32) + bq_ref[...]
    k_ref[...] = lax.dot_general(x, wk_ref[...], dn,
                                 preferred_element_type=jnp.float32)
    v_ref[...] = lax.dot_general(x, wv_ref[...], dn,
                                 preferred_element_type=jnp.float32)


def _project(x, wq_p, bq_p, wk_p, wv_p):
    f32 = jnp.float32
    full = lambda s: pl.BlockSpec(s, lambda i: (0, 0))
    row = pl.BlockSpec((ROW_BLK, HID), lambda i: (i, 0))
    return pl.pallas_call(
        _proj_body,
        grid=(N // ROW_BLK,),
        in_specs=[row, full((HID, HID)), full((1, HID)), full((HID, HID)),
                  full((HID, HID))],
        out_specs=[row, row, row],
        out_shape=[jax.ShapeDtypeStruct((N, HID), f32)] * 3,
    )(x, wq_p, bq_p, wk_p, wv_p)


def _sc_edges(q, k, v, edge_index):
    f32 = jnp.float32
    HB = B // 2
    mesh = plsc.VectorSubcoreMesh(core_axis_name="c", subcore_axis_name="s")

    @functools.partial(
        pl.kernel,
        out_type=[jax.ShapeDtypeStruct((NC, NPAD, HID), f32),
                  jax.ShapeDtypeStruct((NC, ZP_ROWS, HID), f32)],
        mesh=mesh,
        scratch_types=[
            pltpu.VMEM((B,), jnp.int32),
            pltpu.VMEM((B,), jnp.int32),
            pltpu.VMEM((B,), jnp.int32),
            pltpu.VMEM((B,), jnp.int32),
            pltpu.VMEM((B,), jnp.int32),
            pltpu.VMEM((B, HID), f32),
            pltpu.VMEM((B, HID), f32),
            pltpu.VMEM((B, HID), f32),
            pltpu.VMEM((B, 16), f32),
            pltpu.VMEM_SHARED((NPAD, HID), f32),
            pltpu.VMEM_SHARED((ZP_ROWS, HID), f32),
            pltpu.SemaphoreType.DMA,
            pltpu.SemaphoreType.DMA,
            pltpu.SemaphoreType.DMA,
            pltpu.SemaphoreType.DMA,
            pltpu.SemaphoreType.DMA,
            pltpu.SemaphoreType.DMA,
        ],
    )
    def sc_kernel(q_hbm, k_hbm, v_hbm, src_hbm, dst_hbm, wv_hbm, z_hbm,
                  src0, dst0, src1, dst1, zrid, kb, vb, qb, scb,
                  wv_sh, zp_sh, si0, si1, sgA, sgB, sw, sz):
        cid = lax.axis_index("c")
        sid = lax.axis_index("s")
        wid = sid * NC + cid

        zero16 = jnp.zeros((16,), f32)

        @pl.loop(0, B)
        def _zero_fill(r):
            for c in range(8):
                qb[r, pl.ds(c * 16, 16)] = zero16

        @pl.loop(0, ROWS_PER_SUB // 8)
        def _zero_wv(j):
            pltpu.async_copy(qb.at[pl.ds(0, 8)],
                             wv_sh.at[pl.ds(sid * ROWS_PER_SUB + j * 8, 8)],
                             si0)

        @pl.loop(0, ZP_PER_SUB // 8)
        def _zero_zp(j):
            pltpu.async_copy(qb.at[pl.ds(0, 8)],
                             zp_sh.at[pl.ds(sid * ZP_PER_SUB + j * 8, 8)],
                             si0)

        @pl.loop(0, ROWS_PER_SUB // 8 + ZP_PER_SUB // 8)
        def _zero_drain(j):
            pltpu.make_async_copy(
                qb.at[pl.ds(0, 8)],
                wv_sh.at[pl.ds(0, 8)], si0).wait()

        plsc.subcore_barrier()

        def load_idx(c, srcb, dstb, sem):
            off = c * B
            pltpu.async_copy(src_hbm.at[pl.ds(off, B)], srcb, sem)
            pltpu.async_copy(dst_hbm.at[pl.ds(off, B)], dstb, sem)

        def wait_idx(srcb, dstb, sem):
            pltpu.make_async_copy(src_hbm.at[pl.ds(0, B)], srcb, sem).wait()
            pltpu.make_async_copy(dst_hbm.at[pl.ds(0, B)], dstb, sem).wait()

        def process(c_next, srcb, dstb, srcn, dstn, semn, first, last):
            @pl.when(jnp.logical_not(first))
            def _drain_prev():
                pltpu.make_async_copy(qb, wv_sh.at[zrid], sw).wait()
                pltpu.make_async_copy(vb, zp_sh.at[zrid], sz).wait()

            for (tab, idxb, dest) in ((k_hbm, srcb, kb), (v_hbm, srcb, vb),
                                      (q_hbm, dstb, qb)):
                for hh in range(2):
                    pltpu.async_copy(
                        tab.at[idxb.at[pl.ds(hh * HB, HB)]],
                        dest.at[pl.ds(hh * HB, HB)],
                        sgA if hh == 0 else sgB)
            @pl.when(c_next < NCHUNKS)
            def _pref():
                load_idx(c_next, srcn, dstn, semn)

            def edge_loop(lo, hi):
                @pl.loop(lo, hi)
                def _edge(i):
                    acc = kb[i, pl.ds(0, 16)] * qb[i, pl.ds(0, 16)]
                    for c in range(1, 8):
                        acc += (kb[i, pl.ds(c * 16, 16)]
                                * qb[i, pl.ds(c * 16, 16)])
                    ts = (acc + lax.rev(acc, (0,))) * 0.25
                    ts = jnp.minimum(jnp.maximum(ts, -5.0), 5.0)
                    s = jnp.exp(ts)
                    scb[i, :] = s
                    for c in range(8):
                        qb[i, pl.ds(c * 16, 16)] = (
                            vb[i, pl.ds(c * 16, 16)] * s)

            for _ in range(3):
                pltpu.make_async_copy(
                    k_hbm.at[src0.at[pl.ds(0, HB)]],
                    kb.at[pl.ds(0, HB)], sgA).wait()
            edge_loop(0, HB)
            for _ in range(3):
                pltpu.make_async_copy(
                    k_hbm.at[src0.at[pl.ds(0, HB)]],
                    kb.at[pl.ds(0, HB)], sgB).wait()
            edge_loop(HB, B)

            pltpu.async_copy(qb, wv_sh.at[dstb], sw, add=True)

            @pl.loop(0, G)
            def _z_group(g):
                d16 = dstb[pl.ds(g * 16, 16)]
                zrid[pl.ds(g * 16, 16)] = lax.shift_right_logical(d16, 3)
                for t in range(16):
                    i = g * 16 + t
                    s = scb[i, :]
                    for c in range(8):
                        vb[i, pl.ds(c * 16, 16)] = zero16
                    m = lax.rem(d16[t], 8)
                    vb[i, pl.ds(m * 16, 16)] = s

            pltpu.async_copy(vb, zp_sh.at[zrid], sz)

            @pl.when(last)
            def _drain_last():
                pltpu.make_async_copy(qb, wv_sh.at[zrid], sw).wait()
                pltpu.make_async_copy(vb, zp_sh.at[zrid], sz).wait()

        @pl.when(wid < NCHUNKS)
        def _prime():
            load_idx(wid, src0, dst0, si0)

        @pl.loop(0, PAIRS)
        def _pair(jj):
            j0 = jj * 2
            c0 = wid + j0 * NW
            c1 = wid + (j0 + 1) * NW
            c2 = wid + (j0 + 2) * NW

            @pl.when(c0 < NCHUNKS)
            def _proc0():
                wait_idx(src0, dst0, si0)
                process(c1, src0, dst0, src1, dst1, si1,
                        jj == 0, c1 >= NCHUNKS)

            @pl.when(c1 < NCHUNKS)
            def _proc1():
                wait_idx(src1, dst1, si1)
                process(c2, src1, dst1, src0, dst0, si0,
                        jnp.bool_(False), c2 >= NCHUNKS)

        plsc.subcore_barrier()

        base = sid * ROWS_PER_SUB
        pltpu.sync_copy(wv_sh.at[pl.ds(base, ROWS_PER_SUB)],
                        wv_hbm.at[cid, pl.ds(base, ROWS_PER_SUB)])
        zbase = sid * ZP_PER_SUB
        pltpu.sync_copy(zp_sh.at[pl.ds(zbase, ZP_PER_SUB)],
                        z_hbm.at[cid, pl.ds(zbase, ZP_PER_SUB)])

    return sc_kernel(q, k, v, edge_index[0], edge_index[1])


def _out_body(wv0_ref, wv1_ref, z0_ref, z1_ref, t_ref, wo_ref, bo_ref, out_ref):
    wv = wv0_ref[...] + wv1_ref[...]
    z = z0_ref[...] + z1_ref[...]
    den = lax.dot_general(z, t_ref[...], (((1,), (0,)), ((), ())),
                          preferred_element_type=jnp.float32)
    o = wv / den
    out_ref[...] = lax.dot_general(o, wo_ref[...], (((1,), (1,)), ((), ())),
                                   preferred_element_type=jnp.float32) + bo_ref[...]


def _finish(wv0, wv1, z0, z1, tile16, wo_p, bo2):
    f32 = jnp.float32
    row = pl.BlockSpec((ROW_BLK, HID), lambda i: (i, 0))
    zrow = pl.BlockSpec((ROW_BLK, 16), lambda i: (i, 0))
    full = lambda s: pl.BlockSpec(s, lambda i: (0, 0))
    return pl.pallas_call(
        _out_body,
        grid=(N // ROW_BLK,),
        in_specs=[row, row, zrow, zrow, full((16, HID)), full((OUT, HID)),
                  full((1, OUT))],
        out_specs=pl.BlockSpec((ROW_BLK, OUT), lambda i: (i, 0)),
        out_shape=jax.ShapeDtypeStruct((N, OUT), f32),
    )(wv0, wv1, z0, z1, tile16, wo_p, bo2)


def kernel(inputs, edge_index, Wq, bq, Wk, Wv, Wo, bo):
    perm = jnp.asarray(_PERM)
    wq_p = Wq[perm]
    bq_p = bq[perm].reshape(1, HID)
    wk_p = Wk[perm]
    wv_p = Wv[perm]
    wo_p = Wo[:, perm]
    tile16 = jnp.asarray(_TILE16)

    q, k, v = _project(inputs, wq_p, bq_p, wk_p, wv_p)
    wv_parts, z_parts = _sc_edges(q, k, v, edge_index)
    z_flat = z_parts.reshape(NC, ZPAD, 16)
    out = _finish(wv_parts[0], wv_parts[1], z_flat[0], z_flat[1],
                  tile16, wo_p, bo.reshape(1, OUT))
    return out
